# baseline (speedup 1.0000x reference)
import functools

import jax
import jax.numpy as jnp
from jax import lax
from jax.experimental import pallas as pl
from jax.experimental.pallas import tpu as pltpu
from jax.experimental.pallas import tpu_sc as plsc

NUM_FRAMES = 4096
NUM_AUDIO = 8192
AUDIO_DIM = 1280
VIS_DIM = 512
OUT_DIM = VIS_DIM + AUDIO_DIM

FB = 64
CW = 128
NCH = NUM_AUDIO // CW

PART = 2048
NPARTS = NUM_FRAMES // PART

NC = 2
NS = 16
NW = NC * NS
ROWS_W_V = NUM_FRAMES // NW
ROWS_W_G = PART // NW
GCHUNK = 32


def _sc_mesh():
    return plsc.VectorSubcoreMesh(
        core_axis_name="c", subcore_axis_name="s", num_cores=NC, num_subcores=NS
    )


def _argmin_body(f_ref, a_ref, o_ref):
    f = f_ref[...]
    fb = jnp.broadcast_to(f, (FB, CW))

    mv = jnp.abs(a_ref[0:1, :] - fb)
    mc = jnp.zeros((FB, CW), jnp.float32)
    for c in range(1, NCH):
        d = jnp.abs(a_ref[c:c + 1, :] - fb)
        lt = d < mv
        mv = jnp.where(lt, d, mv)
        mc = jnp.where(lt, jnp.float32(c), mc)

    gmin = jnp.min(mv, axis=1, keepdims=True)
    lane = lax.broadcasted_iota(jnp.int32, (FB, CW), 1).astype(jnp.float32)
    gidx = mc * CW + lane
    cand = jnp.where(mv == gmin, gidx, jnp.float32(NUM_AUDIO))
    o_ref[0, 0, :] = jnp.min(cand, axis=1).astype(jnp.int32)


def _closest_idx(frame_ts_part, audio_ts2):
    n = frame_ts_part.shape[0]
    nb = n // FB
    f2 = frame_ts_part.reshape(n, 1)
    idx3 = pl.pallas_call(
        _argmin_body,
        grid=(nb,),
        in_specs=[
            pl.BlockSpec((FB, 1), lambda i: (i, 0)),
            pl.BlockSpec((NCH, CW), lambda i: (0, 0)),
        ],
        out_specs=pl.BlockSpec((1, 1, FB), lambda i: (i, 0, 0)),
        out_shape=jax.ShapeDtypeStruct((nb, 1, FB), jnp.int32),
    )(f2, audio_ts2)
    return idx3.reshape(n)


def _sc_vis_body(vis_hbm, out_hbm):
    wid = lax.axis_index("s") * NC + lax.axis_index("c")
    base = wid * ROWS_W_V
    pltpu.sync_copy(
        vis_hbm.at[pl.ds(base, ROWS_W_V)],
        out_hbm.at[pl.ds(base, ROWS_W_V), pl.ds(0, VIS_DIM)],
    )


@functools.lru_cache(maxsize=1)
def _build_sc_vis():
    return pl.kernel(_sc_vis_body, out_type=(), mesh=_sc_mesh(), scratch_types=[])


def _make_gather_body(start):
    n_chunks = ROWS_W_G // GCHUNK
    assert n_chunks == 2

    def body(audio_hbm, idx_hbm, out_hbm, idx_v, buf_a, buf_b, sem_a, sem_b):
        wid = lax.axis_index("s") * NC + lax.axis_index("c")
        rbase = wid * ROWS_W_G
        obase = start + rbase
        pltpu.sync_copy(idx_hbm.at[pl.ds(rbase, ROWS_W_G)], idx_v)
        bufs = (buf_a, buf_b)
        sems = (sem_a, sem_b)
        copies = [
            pltpu.async_copy(
                audio_hbm.at[idx_v.at[pl.ds(c * GCHUNK, GCHUNK)]], bufs[c], sems[c]
            )
            for c in range(n_chunks)
        ]
        for c in range(n_chunks):
            copies[c].wait()
            pltpu.sync_copy(
                bufs[c],
                out_hbm.at[pl.ds(obase + c * GCHUNK, GCHUNK), pl.ds(VIS_DIM, AUDIO_DIM)],
            )

    return body


@functools.lru_cache(maxsize=None)
def _build_sc_gather(start):
    return pl.kernel(
        _make_gather_body(start),
        out_type=(),
        mesh=_sc_mesh(),
        scratch_types=[
            pltpu.VMEM((ROWS_W_G,), jnp.int32),
            pltpu.VMEM((GCHUNK, AUDIO_DIM), jnp.float32),
            pltpu.VMEM((GCHUNK, AUDIO_DIM), jnp.float32),
            pltpu.SemaphoreType.DMA,
            pltpu.SemaphoreType.DMA,
        ],
    )


def kernel(visual_features, audio_features, audio_timestamps, frame_timestamps):
    a2 = audio_timestamps.reshape(NCH, CW)
    out_ref = jax.empty_ref(
        jax.ShapeDtypeStruct((NUM_FRAMES, OUT_DIM), jnp.float32)
    )
    _build_sc_vis()(visual_features, out_ref)
    for p in range(NPARTS):
        fpart = lax.slice(frame_timestamps, (p * PART,), ((p + 1) * PART,))
        idx = _closest_idx(fpart, a2)
        _build_sc_gather(p * PART)(audio_features, idx, out_ref)
    return out_ref[...]


# speedup vs baseline: 3.5581x; 3.5581x over previous
import functools

import jax
import jax.numpy as jnp
from jax import lax
from jax.experimental import pallas as pl
from jax.experimental.pallas import tpu as pltpu
from jax.experimental.pallas import tpu_sc as plsc

NUM_FRAMES = 4096
NUM_AUDIO = 8192
AUDIO_DIM = 1280
VIS_DIM = 512
OUT_DIM = VIS_DIM + AUDIO_DIM

FB = 64
CW = 128
NCH = NUM_AUDIO // CW

PART = 4096
NPARTS = NUM_FRAMES // PART

NC = 2
NS = 16
NW = NC * NS
ROWS_W_V = NUM_FRAMES // NW
ROWS_W_G = PART // NW
GCHUNK = 32


def _sc_mesh():
    return plsc.VectorSubcoreMesh(
        core_axis_name="c", subcore_axis_name="s", num_cores=NC, num_subcores=NS
    )


def _argmin_body(f_ref, a_ref, o_ref):
    f = f_ref[...]
    fb = jnp.broadcast_to(f, (FB, CW))

    mv = jnp.abs(a_ref[0:1, :] - fb)
    mc = jnp.zeros((FB, CW), jnp.float32)
    for c in range(1, NCH):
        d = jnp.abs(a_ref[c:c + 1, :] - fb)
        lt = d < mv
        mv = jnp.where(lt, d, mv)
        mc = jnp.where(lt, jnp.float32(c), mc)

    gmin = jnp.min(mv, axis=1, keepdims=True)
    lane = lax.broadcasted_iota(jnp.int32, (FB, CW), 1).astype(jnp.float32)
    gidx = mc * CW + lane
    cand = jnp.where(mv == gmin, gidx, jnp.float32(NUM_AUDIO))
    o_ref[0, 0, :] = jnp.min(cand, axis=1).astype(jnp.int32)


def _closest_idx(frame_ts_part, audio_ts2):
    n = frame_ts_part.shape[0]
    nb = n // FB
    f2 = frame_ts_part.reshape(n, 1)
    idx3 = pl.pallas_call(
        _argmin_body,
        grid=(nb,),
        in_specs=[
            pl.BlockSpec((FB, 1), lambda i: (i, 0)),
            pl.BlockSpec((NCH, CW), lambda i: (0, 0)),
        ],
        out_specs=pl.BlockSpec((1, 1, FB), lambda i: (i, 0, 0)),
        out_shape=jax.ShapeDtypeStruct((nb, 1, FB), jnp.int32),
    )(f2, audio_ts2)
    return idx3.reshape(n)


VCHUNK = 32
N_VCH = ROWS_W_V // VCHUNK


def _sc_vis_body(vis_hbm, out_hbm, buf_a, buf_b, sem_a, sem_b):
    wid = lax.axis_index("s") * NC + lax.axis_index("c")
    base = wid * ROWS_W_V
    bufs = (buf_a, buf_b)
    sems = (sem_a, sem_b)
    copies = [None] * N_VCH
    copies[0] = pltpu.async_copy(vis_hbm.at[pl.ds(base, VCHUNK)], bufs[0], sems[0])
    for c in range(N_VCH):
        if c + 1 < N_VCH:
            copies[c + 1] = pltpu.async_copy(
                vis_hbm.at[pl.ds(base + (c + 1) * VCHUNK, VCHUNK)],
                bufs[(c + 1) % 2],
                sems[(c + 1) % 2],
            )
        copies[c].wait()
        pltpu.sync_copy(
            bufs[c % 2],
            out_hbm.at[pl.ds(base + c * VCHUNK, VCHUNK), pl.ds(0, VIS_DIM)],
        )


@functools.lru_cache(maxsize=1)
def _build_sc_vis():
    return pl.kernel(
        _sc_vis_body,
        out_type=(),
        mesh=_sc_mesh(),
        scratch_types=[
            pltpu.VMEM((VCHUNK, VIS_DIM), jnp.float32),
            pltpu.VMEM((VCHUNK, VIS_DIM), jnp.float32),
            pltpu.SemaphoreType.DMA,
            pltpu.SemaphoreType.DMA,
        ],
    )


def _make_gather_body(start):
    n_chunks = ROWS_W_G // GCHUNK

    def body(audio_hbm, idx_hbm, out_hbm, idx_v, buf_a, buf_b, sem_a, sem_b):
        wid = lax.axis_index("s") * NC + lax.axis_index("c")
        rbase = wid * ROWS_W_G
        obase = start + rbase
        pltpu.sync_copy(idx_hbm.at[pl.ds(rbase, ROWS_W_G)], idx_v)
        bufs = (buf_a, buf_b)
        sems = (sem_a, sem_b)
        copies = [None] * n_chunks
        copies[0] = pltpu.async_copy(
            audio_hbm.at[idx_v.at[pl.ds(0, GCHUNK)]], bufs[0], sems[0]
        )
        for c in range(n_chunks):
            if c + 1 < n_chunks:
                copies[c + 1] = pltpu.async_copy(
                    audio_hbm.at[idx_v.at[pl.ds((c + 1) * GCHUNK, GCHUNK)]],
                    bufs[(c + 1) % 2],
                    sems[(c + 1) % 2],
                )
            copies[c].wait()
            pltpu.sync_copy(
                bufs[c % 2],
                out_hbm.at[pl.ds(obase + c * GCHUNK, GCHUNK), pl.ds(VIS_DIM, AUDIO_DIM)],
            )

    return body


@functools.lru_cache(maxsize=None)
def _build_sc_gather(start):
    return pl.kernel(
        _make_gather_body(start),
        out_type=(),
        mesh=_sc_mesh(),
        scratch_types=[
            pltpu.VMEM((ROWS_W_G,), jnp.int32),
            pltpu.VMEM((GCHUNK, AUDIO_DIM), jnp.float32),
            pltpu.VMEM((GCHUNK, AUDIO_DIM), jnp.float32),
            pltpu.SemaphoreType.DMA,
            pltpu.SemaphoreType.DMA,
        ],
    )


def kernel(visual_features, audio_features, audio_timestamps, frame_timestamps):
    a2 = audio_timestamps.reshape(NCH, CW)
    out_ref = jax.empty_ref(
        jax.ShapeDtypeStruct((NUM_FRAMES, OUT_DIM), jnp.float32)
    )
    _build_sc_vis()(visual_features, out_ref)
    for p in range(NPARTS):
        fpart = lax.slice(frame_timestamps, (p * PART,), ((p + 1) * PART,))
        idx = _closest_idx(fpart, a2)
        _build_sc_gather(p * PART)(audio_features, idx, out_ref)
    return out_ref[...]


# speedup vs baseline: 3.9485x; 1.1097x over previous
import functools

import jax
import jax.numpy as jnp
from jax import lax
from jax.experimental import pallas as pl
from jax.experimental.pallas import tpu as pltpu
from jax.experimental.pallas import tpu_sc as plsc

NUM_FRAMES = 4096
NUM_AUDIO = 8192
AUDIO_DIM = 1280
VIS_DIM = 512
OUT_DIM = VIS_DIM + AUDIO_DIM

FB = 64
CW = 128
NCH = NUM_AUDIO // CW

PART = 4096
NPARTS = NUM_FRAMES // PART

NC = 2
NS = 16
NW = NC * NS
ROWS_W_V = NUM_FRAMES // NW
ROWS_W_G = PART // NW
GCHUNK = 32


def _sc_mesh():
    return plsc.VectorSubcoreMesh(
        core_axis_name="c", subcore_axis_name="s", num_cores=NC, num_subcores=NS
    )


def _argmin_body(f_ref, a_ref, o_ref):
    nb = f_ref.shape[0] // FB

    def blk(i, carry):
        f = f_ref[pl.ds(i * FB, FB), :]
        fb = jnp.broadcast_to(f, (FB, CW))
        mv = jnp.abs(a_ref[0:1, :] - fb)
        mc = jnp.zeros((FB, CW), jnp.float32)
        for c in range(1, NCH):
            d = jnp.abs(a_ref[c:c + 1, :] - fb)
            lt = d < mv
            mv = jnp.where(lt, d, mv)
            mc = jnp.where(lt, jnp.float32(c), mc)
        gmin = jnp.min(mv, axis=1, keepdims=True)
        lane = lax.broadcasted_iota(jnp.int32, (FB, CW), 1).astype(jnp.float32)
        gidx = mc * CW + lane
        cand = jnp.where(mv == gmin, gidx, jnp.float32(NUM_AUDIO))
        o_ref[pl.ds(i, 1), :, :] = jnp.min(cand, axis=1).astype(jnp.int32).reshape(1, 1, FB)
        return carry

    lax.fori_loop(0, nb, blk, 0)


def _closest_idx(frame_ts_part, audio_ts2):
    n = frame_ts_part.shape[0]
    nb = n // FB
    f2 = frame_ts_part.reshape(n, 1)
    idx3 = pl.pallas_call(
        _argmin_body,
        out_shape=jax.ShapeDtypeStruct((nb, 1, FB), jnp.int32),
    )(f2, audio_ts2)
    return idx3.reshape(n)


VCHUNK = 32
N_VCH = ROWS_W_V // VCHUNK


def _sc_vis_body(vis_hbm, out_hbm, buf_a, buf_b, sem_a, sem_b):
    wid = lax.axis_index("s") * NC + lax.axis_index("c")
    base = wid * ROWS_W_V
    bufs = (buf_a, buf_b)
    sems = (sem_a, sem_b)
    copies = [None] * N_VCH
    copies[0] = pltpu.async_copy(vis_hbm.at[pl.ds(base, VCHUNK)], bufs[0], sems[0])
    for c in range(N_VCH):
        if c + 1 < N_VCH:
            copies[c + 1] = pltpu.async_copy(
                vis_hbm.at[pl.ds(base + (c + 1) * VCHUNK, VCHUNK)],
                bufs[(c + 1) % 2],
                sems[(c + 1) % 2],
            )
        copies[c].wait()
        pltpu.sync_copy(
            bufs[c % 2],
            out_hbm.at[pl.ds(base + c * VCHUNK, VCHUNK), pl.ds(0, VIS_DIM)],
        )


@functools.lru_cache(maxsize=1)
def _build_sc_vis():
    return pl.kernel(
        _sc_vis_body,
        out_type=(),
        mesh=_sc_mesh(),
        scratch_types=[
            pltpu.VMEM((VCHUNK, VIS_DIM), jnp.float32),
            pltpu.VMEM((VCHUNK, VIS_DIM), jnp.float32),
            pltpu.SemaphoreType.DMA,
            pltpu.SemaphoreType.DMA,
        ],
    )


def _make_gather_body(start):
    n_chunks = ROWS_W_G // GCHUNK

    def body(audio_hbm, idx_hbm, out_hbm, idx_v, buf_a, buf_b, sem_a, sem_b):
        wid = lax.axis_index("s") * NC + lax.axis_index("c")
        rbase = wid * ROWS_W_G
        obase = start + rbase
        pltpu.sync_copy(idx_hbm.at[pl.ds(rbase, ROWS_W_G)], idx_v)
        bufs = (buf_a, buf_b)
        sems = (sem_a, sem_b)
        copies = [None] * n_chunks
        copies[0] = pltpu.async_copy(
            audio_hbm.at[idx_v.at[pl.ds(0, GCHUNK)]], bufs[0], sems[0]
        )
        for c in range(n_chunks):
            if c + 1 < n_chunks:
                copies[c + 1] = pltpu.async_copy(
                    audio_hbm.at[idx_v.at[pl.ds((c + 1) * GCHUNK, GCHUNK)]],
                    bufs[(c + 1) % 2],
                    sems[(c + 1) % 2],
                )
            copies[c].wait()
            pltpu.sync_copy(
                bufs[c % 2],
                out_hbm.at[pl.ds(obase + c * GCHUNK, GCHUNK), pl.ds(VIS_DIM, AUDIO_DIM)],
            )

    return body


@functools.lru_cache(maxsize=None)
def _build_sc_gather(start):
    return pl.kernel(
        _make_gather_body(start),
        out_type=(),
        mesh=_sc_mesh(),
        scratch_types=[
            pltpu.VMEM((ROWS_W_G,), jnp.int32),
            pltpu.VMEM((GCHUNK, AUDIO_DIM), jnp.float32),
            pltpu.VMEM((GCHUNK, AUDIO_DIM), jnp.float32),
            pltpu.SemaphoreType.DMA,
            pltpu.SemaphoreType.DMA,
        ],
    )


def kernel(visual_features, audio_features, audio_timestamps, frame_timestamps):
    a2 = audio_timestamps.reshape(NCH, CW)
    out_ref = jax.empty_ref(
        jax.ShapeDtypeStruct((NUM_FRAMES, OUT_DIM), jnp.float32)
    )
    _build_sc_vis()(visual_features, out_ref)
    for p in range(NPARTS):
        fpart = lax.slice(frame_timestamps, (p * PART,), ((p + 1) * PART,))
        idx = _closest_idx(fpart, a2)
        _build_sc_gather(p * PART)(audio_features, idx, out_ref)
    return out_ref[...]


# speedup vs baseline: 4.1805x; 1.0587x over previous
import functools

import jax
import jax.numpy as jnp
from jax import lax
from jax.experimental import pallas as pl
from jax.experimental.pallas import tpu as pltpu
from jax.experimental.pallas import tpu_sc as plsc

NUM_FRAMES = 4096
NUM_AUDIO = 8192
AUDIO_DIM = 1280
VIS_DIM = 512
OUT_DIM = VIS_DIM + AUDIO_DIM

FB = 64
CW = 128
NCH = NUM_AUDIO // CW

PART = 2048
NPARTS = NUM_FRAMES // PART

NC = 2
NS = 16
NW = NC * NS
ROWS_W_V = NUM_FRAMES // NW
ROWS_W_G = PART // NW
GCHUNK = 32


def _sc_mesh():
    return plsc.VectorSubcoreMesh(
        core_axis_name="c", subcore_axis_name="s", num_cores=NC, num_subcores=NS
    )


def _argmin_body(f_ref, a_ref, o_ref):
    nb = f_ref.shape[0] // FB

    def blk(i, carry):
        f = f_ref[pl.ds(i * FB, FB), :]
        fb = jnp.broadcast_to(f, (FB, CW))
        mv = jnp.abs(a_ref[0:1, :] - fb)
        mc = jnp.zeros((FB, CW), jnp.float32)
        for c in range(1, NCH):
            d = jnp.abs(a_ref[c:c + 1, :] - fb)
            lt = d < mv
            mv = jnp.where(lt, d, mv)
            mc = jnp.where(lt, jnp.float32(c), mc)
        gmin = jnp.min(mv, axis=1, keepdims=True)
        lane = lax.broadcasted_iota(jnp.int32, (FB, CW), 1).astype(jnp.float32)
        gidx = mc * CW + lane
        cand = jnp.where(mv == gmin, gidx, jnp.float32(NUM_AUDIO))
        o_ref[pl.ds(i, 1), :, :] = jnp.min(cand, axis=1).astype(jnp.int32).reshape(1, 1, FB)
        return carry

    lax.fori_loop(0, nb, blk, 0)


def _closest_idx(frame_ts_part, audio_ts2):
    n = frame_ts_part.shape[0]
    nb = n // FB
    f2 = frame_ts_part.reshape(n, 1)
    idx3 = pl.pallas_call(
        _argmin_body,
        out_shape=jax.ShapeDtypeStruct((nb, 1, FB), jnp.int32),
    )(f2, audio_ts2)
    return idx3.reshape(n)


VCHUNK = 32
N_VCH = ROWS_W_V // VCHUNK


def _sc_vis_body(vis_hbm, out_hbm, buf_a, buf_b, sem_a, sem_b):
    wid = lax.axis_index("s") * NC + lax.axis_index("c")
    base = wid * ROWS_W_V
    bufs = (buf_a, buf_b)
    sems = (sem_a, sem_b)
    copies = [None] * N_VCH
    copies[0] = pltpu.async_copy(vis_hbm.at[pl.ds(base, VCHUNK)], bufs[0], sems[0])
    for c in range(N_VCH):
        if c + 1 < N_VCH:
            copies[c + 1] = pltpu.async_copy(
                vis_hbm.at[pl.ds(base + (c + 1) * VCHUNK, VCHUNK)],
                bufs[(c + 1) % 2],
                sems[(c + 1) % 2],
            )
        copies[c].wait()
        pltpu.sync_copy(
            bufs[c % 2],
            out_hbm.at[pl.ds(base + c * VCHUNK, VCHUNK), pl.ds(0, VIS_DIM)],
        )


@functools.lru_cache(maxsize=1)
def _build_sc_vis():
    return pl.kernel(
        _sc_vis_body,
        out_type=(),
        mesh=_sc_mesh(),
        scratch_types=[
            pltpu.VMEM((VCHUNK, VIS_DIM), jnp.float32),
            pltpu.VMEM((VCHUNK, VIS_DIM), jnp.float32),
            pltpu.SemaphoreType.DMA,
            pltpu.SemaphoreType.DMA,
        ],
    )


def _make_gather_body(start):
    n_chunks = ROWS_W_G // GCHUNK

    def body(audio_hbm, idx_hbm, out_hbm, idx_v, buf_a, buf_b, sem_a, sem_b):
        wid = lax.axis_index("s") * NC + lax.axis_index("c")
        rbase = wid * ROWS_W_G
        obase = start + rbase
        pltpu.sync_copy(idx_hbm.at[pl.ds(rbase, ROWS_W_G)], idx_v)
        bufs = (buf_a, buf_b)
        sems = (sem_a, sem_b)
        copies = [None] * n_chunks
        copies[0] = pltpu.async_copy(
            audio_hbm.at[idx_v.at[pl.ds(0, GCHUNK)]], bufs[0], sems[0]
        )
        for c in range(n_chunks):
            if c + 1 < n_chunks:
                copies[c + 1] = pltpu.async_copy(
                    audio_hbm.at[idx_v.at[pl.ds((c + 1) * GCHUNK, GCHUNK)]],
                    bufs[(c + 1) % 2],
                    sems[(c + 1) % 2],
                )
            copies[c].wait()
            pltpu.sync_copy(
                bufs[c % 2],
                out_hbm.at[pl.ds(obase + c * GCHUNK, GCHUNK), pl.ds(VIS_DIM, AUDIO_DIM)],
            )

    return body


@functools.lru_cache(maxsize=None)
def _build_sc_gather(start):
    return pl.kernel(
        _make_gather_body(start),
        out_type=(),
        mesh=_sc_mesh(),
        scratch_types=[
            pltpu.VMEM((ROWS_W_G,), jnp.int32),
            pltpu.VMEM((GCHUNK, AUDIO_DIM), jnp.float32),
            pltpu.VMEM((GCHUNK, AUDIO_DIM), jnp.float32),
            pltpu.SemaphoreType.DMA,
            pltpu.SemaphoreType.DMA,
        ],
    )


def kernel(visual_features, audio_features, audio_timestamps, frame_timestamps):
    a2 = audio_timestamps.reshape(NCH, CW)
    out_ref = jax.empty_ref(
        jax.ShapeDtypeStruct((NUM_FRAMES, OUT_DIM), jnp.float32)
    )
    _build_sc_vis()(visual_features, out_ref)
    for p in range(NPARTS):
        fpart = lax.slice(frame_timestamps, (p * PART,), ((p + 1) * PART,))
        idx = _closest_idx(fpart, a2)
        _build_sc_gather(p * PART)(audio_features, idx, out_ref)
    return out_ref[...]


# speedup vs baseline: 5.4648x; 1.3072x over previous
import functools

import jax
import jax.numpy as jnp
from jax import lax
from jax.experimental import pallas as pl
from jax.experimental.pallas import tpu as pltpu
from jax.experimental.pallas import tpu_sc as plsc

NUM_FRAMES = 4096
NUM_AUDIO = 8192
AUDIO_DIM = 1280
VIS_DIM = 512
OUT_DIM = VIS_DIM + AUDIO_DIM

FB = 2048
CW = 128
NCH = NUM_AUDIO // CW

PART = 2048
NPARTS = NUM_FRAMES // PART

NC = 2
NS = 16
NW = NC * NS
ROWS_W_V = NUM_FRAMES // NW
ROWS_W_G = PART // NW
GCHUNK = 32


def _sc_mesh():
    return plsc.VectorSubcoreMesh(
        core_axis_name="c", subcore_axis_name="s", num_cores=NC, num_subcores=NS
    )


def _argmin_body(f_ref, a_ref, o_ref):
    nb = f_ref.shape[0] // FB

    def blk(i, carry):
        f = f_ref[pl.ds(i * FB, FB), :]
        fb = jnp.broadcast_to(f, (FB, CW))
        mv = jnp.abs(a_ref[0:1, :] - fb)
        mc = jnp.zeros((FB, CW), jnp.float32)
        for c in range(1, NCH):
            d = jnp.abs(a_ref[c:c + 1, :] - fb)
            lt = d < mv
            mv = jnp.where(lt, d, mv)
            mc = jnp.where(lt, jnp.float32(c), mc)
        gmin = jnp.min(mv, axis=1, keepdims=True)
        lane = lax.broadcasted_iota(jnp.int32, (FB, CW), 1).astype(jnp.float32)
        gidx = mc * CW + lane
        cand = jnp.where(mv == gmin, gidx, jnp.float32(NUM_AUDIO))
        o_ref[pl.ds(i, 1), :, :] = jnp.min(cand, axis=1).astype(jnp.int32).reshape(1, 1, FB)
        return carry

    lax.fori_loop(0, nb, blk, 0)


def _closest_idx(frame_ts_part, audio_ts2):
    n = frame_ts_part.shape[0]
    nb = n // FB
    f2 = frame_ts_part.reshape(n, 1)
    idx3 = pl.pallas_call(
        _argmin_body,
        out_shape=jax.ShapeDtypeStruct((nb, 1, FB), jnp.int32),
    )(f2, audio_ts2)
    return idx3.reshape(n)


VCHUNK = 32
N_VCH = ROWS_W_V // VCHUNK


def _sc_vis_body(vis_hbm, out_hbm, buf_a, buf_b, sem_a, sem_b):
    wid = lax.axis_index("s") * NC + lax.axis_index("c")
    base = wid * ROWS_W_V
    bufs = (buf_a, buf_b)
    sems = (sem_a, sem_b)
    copies = [None] * N_VCH
    copies[0] = pltpu.async_copy(vis_hbm.at[pl.ds(base, VCHUNK)], bufs[0], sems[0])
    for c in range(N_VCH):
        if c + 1 < N_VCH:
            copies[c + 1] = pltpu.async_copy(
                vis_hbm.at[pl.ds(base + (c + 1) * VCHUNK, VCHUNK)],
                bufs[(c + 1) % 2],
                sems[(c + 1) % 2],
            )
        copies[c].wait()
        pltpu.sync_copy(
            bufs[c % 2],
            out_hbm.at[pl.ds(base + c * VCHUNK, VCHUNK), pl.ds(0, VIS_DIM)],
        )


@functools.lru_cache(maxsize=1)
def _build_sc_vis():
    return pl.kernel(
        _sc_vis_body,
        out_type=(),
        mesh=_sc_mesh(),
        scratch_types=[
            pltpu.VMEM((VCHUNK, VIS_DIM), jnp.float32),
            pltpu.VMEM((VCHUNK, VIS_DIM), jnp.float32),
            pltpu.SemaphoreType.DMA,
            pltpu.SemaphoreType.DMA,
        ],
    )


def _make_gather_body(start):
    n_chunks = ROWS_W_G // GCHUNK

    def body(audio_hbm, idx_hbm, out_hbm, idx_v, buf_a, buf_b, sem_a, sem_b):
        wid = lax.axis_index("s") * NC + lax.axis_index("c")
        rbase = wid * ROWS_W_G
        obase = start + rbase
        pltpu.sync_copy(idx_hbm.at[pl.ds(rbase, ROWS_W_G)], idx_v)
        bufs = (buf_a, buf_b)
        sems = (sem_a, sem_b)
        copies = [None] * n_chunks
        copies[0] = pltpu.async_copy(
            audio_hbm.at[idx_v.at[pl.ds(0, GCHUNK)]], bufs[0], sems[0]
        )
        for c in range(n_chunks):
            if c + 1 < n_chunks:
                copies[c + 1] = pltpu.async_copy(
                    audio_hbm.at[idx_v.at[pl.ds((c + 1) * GCHUNK, GCHUNK)]],
                    bufs[(c + 1) % 2],
                    sems[(c + 1) % 2],
                )
            copies[c].wait()
            pltpu.sync_copy(
                bufs[c % 2],
                out_hbm.at[pl.ds(obase + c * GCHUNK, GCHUNK), pl.ds(VIS_DIM, AUDIO_DIM)],
            )

    return body


@functools.lru_cache(maxsize=None)
def _build_sc_gather(start):
    return pl.kernel(
        _make_gather_body(start),
        out_type=(),
        mesh=_sc_mesh(),
        scratch_types=[
            pltpu.VMEM((ROWS_W_G,), jnp.int32),
            pltpu.VMEM((GCHUNK, AUDIO_DIM), jnp.float32),
            pltpu.VMEM((GCHUNK, AUDIO_DIM), jnp.float32),
            pltpu.SemaphoreType.DMA,
            pltpu.SemaphoreType.DMA,
        ],
    )


def kernel(visual_features, audio_features, audio_timestamps, frame_timestamps):
    a2 = audio_timestamps.reshape(NCH, CW)
    out_ref = jax.empty_ref(
        jax.ShapeDtypeStruct((NUM_FRAMES, OUT_DIM), jnp.float32)
    )
    _build_sc_vis()(visual_features, out_ref)
    for p in range(NPARTS):
        fpart = lax.slice(frame_timestamps, (p * PART,), ((p + 1) * PART,))
        idx = _closest_idx(fpart, a2)
        _build_sc_gather(p * PART)(audio_features, idx, out_ref)
    return out_ref[...]
